# baseline (speedup 1.0000x reference)
import functools

import jax
import jax.numpy as jnp
from jax import lax
from jax.experimental import pallas as pl
from jax.experimental.pallas import tpu as pltpu
from jax.experimental.pallas import tpu_sc as plsc

_N = 96
_B, _HG, _WG = 4, 256, 256
_P = _B * _HG * _WG
_LAYERS = ((512, 512, 0), (256, 256, 262144), (128, 128, 327680), (64, 64, 344064))
_NW = 32
_PTS_PER_W = _P // _NW
_C = 16
_CHUNKS = _PTS_PER_W // _C
_TAPS = 16
_ROWS = _TAPS * _C


def _sc_sample(table, ux, uy):
    mesh = plsc.VectorSubcoreMesh(core_axis_name="c", subcore_axis_name="s")

    @functools.partial(
        pl.kernel,
        out_type=jax.ShapeDtypeStruct((_P * _N,), jnp.float32),
        mesh=mesh,
        compiler_params=pltpu.CompilerParams(use_tc_tiling_on_sc=False),
        scratch_types=[
            pltpu.VMEM((_C,), jnp.float32),
            pltpu.VMEM((_C,), jnp.float32),
            pltpu.VMEM((4 * _C,), jnp.int32),
            pltpu.VMEM((4 * _C,), jnp.int32),
            pltpu.VMEM((4 * _C,), jnp.int32),
            pltpu.VMEM((4 * _C,), jnp.int32),
            pltpu.VMEM((_ROWS,), jnp.float32),
            pltpu.VMEM((_ROWS,), jnp.float32),
            pltpu.VMEM((_ROWS, _N), jnp.float32),
            pltpu.VMEM((_ROWS, _N), jnp.float32),
            pltpu.VMEM((_C * _N,), jnp.float32),
            pltpu.SemaphoreType.DMA,
            pltpu.SemaphoreType.DMA,
        ],
    )
    def tex_kernel(table_hbm, ux_hbm, uy_hbm, out_hbm,
                   ux_v, uy_v, idx0, idx1, idx2, idx3,
                   w_a, w_b, taps_a, taps_b, out_v, sem_a, sem_b):
        idx_refs = (idx0, idx1, idx2, idx3)
        wid = lax.axis_index("s") * 2 + lax.axis_index("c")
        pbase = wid * _PTS_PER_W

        def stage(i, w_v):
            base = pbase + i * _C
            pltpu.sync_copy(ux_hbm.at[pl.ds(base, _C)], ux_v)
            pltpu.sync_copy(uy_hbm.at[pl.ds(base, _C)], uy_v)
            for g in range(_C // 16):
                x = ux_v[pl.ds(g * 16, 16)]
                y = uy_v[pl.ds(g * 16, 16)]
                for l, (h, w, off) in enumerate(_LAYERS):
                    fx = (x + 1.0) * 0.5 * (w - 1)
                    fy = (y + 1.0) * 0.5 * (h - 1)
                    x0 = jnp.minimum(fx.astype(jnp.int32), w - 2)
                    y0 = jnp.minimum(fy.astype(jnp.int32), h - 2)
                    wx1 = fx - x0.astype(jnp.float32)
                    wy1 = fy - y0.astype(jnp.float32)
                    wx0 = 1.0 - wx1
                    wy0 = 1.0 - wy1
                    i00 = (off + y0 * w) + x0
                    idx_refs[l][pl.ds(0 * _C + g * 16, 16)] = i00
                    idx_refs[l][pl.ds(1 * _C + g * 16, 16)] = i00 + 1
                    idx_refs[l][pl.ds(2 * _C + g * 16, 16)] = i00 + w
                    idx_refs[l][pl.ds(3 * _C + g * 16, 16)] = i00 + (w + 1)
                    w_v[pl.ds((l * 4 + 0) * _C + g * 16, 16)] = wy0 * wx0
                    w_v[pl.ds((l * 4 + 1) * _C + g * 16, 16)] = wy0 * wx1
                    w_v[pl.ds((l * 4 + 2) * _C + g * 16, 16)] = wy1 * wx0
                    w_v[pl.ds((l * 4 + 3) * _C + g * 16, 16)] = wy1 * wx1

        def copies(taps_v, sem):
            return [
                pltpu.make_async_copy(table_hbm.at[idx_refs[l]],
                                      taps_v.at[pl.ds(l * 4 * _C, 4 * _C)],
                                      sem)
                for l in range(4)
            ]

        def issue(taps_v, sem):
            for cp in copies(taps_v, sem):
                cp.start()

        def drain(taps_v, sem):
            for cp in copies(taps_v, sem):
                cp.wait()

        def accumulate(i, w_v, taps_v):
            for g in range(_C // 16):
                wvecs = [w_v[pl.ds(t * _C + g * 16, 16)] for t in range(_TAPS)]
                for pp in range(16):
                    p = g * 16 + pp
                    ws = [wvecs[t][pp] for t in range(_TAPS)]
                    for k in range(_N // 16):
                        terms = [ws[t] * taps_v[t * _C + p, pl.ds(k * 16, 16)]
                                 for t in range(_TAPS)]
                        while len(terms) > 1:
                            terms = [terms[j] + terms[j + 1]
                                     for j in range(0, len(terms), 2)]
                        out_v[pl.ds(p * _N + k * 16, 16)] = terms[0]
            pltpu.sync_copy(out_v, out_hbm.at[pl.ds((pbase + i * _C) * _N, _C * _N)])

        stage(0, w_a)
        issue(taps_a, sem_a)

        def body(j, carry):
            ia = 2 * j
            ib = 2 * j + 1
            drain(taps_a, sem_a)
            stage(ib, w_b)
            issue(taps_b, sem_b)
            accumulate(ia, w_a, taps_a)

            drain(taps_b, sem_b)

            @pl.when(ib + 1 < _CHUNKS)
            def _():
                stage(ib + 1, w_a)
                issue(taps_a, sem_a)

            accumulate(ib, w_b, taps_b)
            return carry

        lax.fori_loop(0, _CHUNKS // 2, body, 0)

    return tex_kernel(table, ux, uy)


def kernel(uv, layer1, layer2, layer3, layer4):
    tabs = [l[0].reshape(_N, -1).T for l in (layer1, layer2, layer3, layer4)]
    table = jnp.concatenate(tabs, axis=0)
    ux = uv[..., 0].reshape(-1)
    uy = uv[..., 1].reshape(-1)
    out = _sc_sample(table, ux, uy)
    return out.reshape(_B, _HG, _WG, _N).transpose(0, 3, 1, 2)


# speedup vs baseline: 1.0445x; 1.0445x over previous
import functools

import jax
import jax.numpy as jnp
from jax import lax
from jax.experimental import pallas as pl
from jax.experimental.pallas import tpu as pltpu
from jax.experimental.pallas import tpu_sc as plsc

_N = 96
_B, _HG, _WG = 4, 256, 256
_P = _B * _HG * _WG
_LAYERS = ((512, 512, 0), (256, 256, 262144), (128, 128, 327680), (64, 64, 344064))
_NW = 32
_PTS_PER_W = _P // _NW
_C = 16
_CHUNKS = _PTS_PER_W // _C
_TAPS = 16
_ROWS = _TAPS * _C


def _sc_sample(table, ux, uy):
    mesh = plsc.VectorSubcoreMesh(core_axis_name="c", subcore_axis_name="s")

    @functools.partial(
        pl.kernel,
        out_type=jax.ShapeDtypeStruct((_P * _N,), jnp.float32),
        mesh=mesh,
        compiler_params=pltpu.CompilerParams(use_tc_tiling_on_sc=False),
        scratch_types=[
            pltpu.VMEM((_C,), jnp.float32),
            pltpu.VMEM((_C,), jnp.float32),
            pltpu.VMEM((_ROWS,), jnp.int32),
            pltpu.VMEM((_ROWS,), jnp.int32),
            pltpu.VMEM((_ROWS,), jnp.float32),
            pltpu.VMEM((_ROWS,), jnp.float32),
            pltpu.VMEM((_ROWS, _N), jnp.float32),
            pltpu.VMEM((_ROWS, _N), jnp.float32),
            pltpu.VMEM((_C * _N,), jnp.float32),
            pltpu.SemaphoreType.DMA,
            pltpu.SemaphoreType.DMA,
        ],
    )
    def tex_kernel(table_hbm, ux_hbm, uy_hbm, out_hbm,
                   ux_v, uy_v, idx_a, idx_b, w_a, w_b, taps_a, taps_b,
                   out_v, sem_a, sem_b):
        wid = lax.axis_index("s") * 2 + lax.axis_index("c")
        pbase = wid * _PTS_PER_W

        def stage(i, idx_v, w_v):
            iota = lax.iota(jnp.int32, 16)
            lane_l = jnp.right_shift(iota, 2)
            wpitch_v = jnp.right_shift(iota * 0 + 512, lane_l)
            wm2_v = wpitch_v - 2
            sx_v = (wpitch_v - 1).astype(jnp.float32) * 0.5
            off_v = 349525 - jnp.right_shift(iota * 0 + 349525, 2 * lane_l)
            dx_v = jnp.bitwise_and(iota, 1)
            dy_v = jnp.bitwise_and(jnp.right_shift(iota, 1), 1)
            maskx = dx_v == 1
            masky = dy_v == 1

            base = pbase + i * _C
            pltpu.sync_copy(ux_hbm.at[pl.ds(base, _C)], ux_v)
            pltpu.sync_copy(uy_hbm.at[pl.ds(base, _C)], uy_v)
            xs = ux_v[pl.ds(0, 16)]
            ys = uy_v[pl.ds(0, 16)]
            for p in range(_C):
                fx = (xs[p] + 1.0) * sx_v
                fy = (ys[p] + 1.0) * sx_v
                x0 = jnp.minimum(fx.astype(jnp.int32), wm2_v)
                y0 = jnp.minimum(fy.astype(jnp.int32), wm2_v)
                wx1 = fx - x0.astype(jnp.float32)
                wy1 = fy - y0.astype(jnp.float32)
                wxs = jnp.where(maskx, wx1, 1.0 - wx1)
                wys = jnp.where(masky, wy1, 1.0 - wy1)
                idx_v[pl.ds(p * _TAPS, 16)] = (y0 + dy_v) * wpitch_v + (x0 + dx_v) + off_v
                w_v[pl.ds(p * _TAPS, 16)] = wxs * wys

        def copies(idx_v, taps_v, sem):
            return [
                pltpu.make_async_copy(table_hbm.at[idx_v.at[pl.ds(j * 128, 128)]],
                                      taps_v.at[pl.ds(j * 128, 128)],
                                      sem)
                for j in range(_ROWS // 128)
            ]

        def issue(idx_v, taps_v, sem):
            for cp in copies(idx_v, taps_v, sem):
                cp.start()

        def drain(idx_v, taps_v, sem):
            for cp in copies(idx_v, taps_v, sem):
                cp.wait()

        def accumulate(i, w_v, taps_v):
            for p in range(_C):
                wv = w_v[pl.ds(p * _TAPS, 16)]
                accs = [None] * (_N // 16)
                for t in range(_TAPS):
                    w = wv[t]
                    for k in range(_N // 16):
                        term = w * taps_v[p * _TAPS + t, pl.ds(k * 16, 16)]
                        accs[k] = term if t == 0 else accs[k] + term
                for k in range(_N // 16):
                    out_v[pl.ds(p * _N + k * 16, 16)] = accs[k]
            pltpu.sync_copy(out_v, out_hbm.at[pl.ds((pbase + i * _C) * _N, _C * _N)])

        stage(0, idx_a, w_a)
        issue(idx_a, taps_a, sem_a)

        def body(j, carry):
            ia = 2 * j
            ib = 2 * j + 1
            stage(ib, idx_b, w_b)
            issue(idx_b, taps_b, sem_b)
            drain(idx_a, taps_a, sem_a)
            accumulate(ia, w_a, taps_a)

            @pl.when(ib + 1 < _CHUNKS)
            def _():
                stage(ib + 1, idx_a, w_a)
                issue(idx_a, taps_a, sem_a)

            drain(idx_b, taps_b, sem_b)
            accumulate(ib, w_b, taps_b)
            return carry

        lax.fori_loop(0, _CHUNKS // 2, body, 0)

    return tex_kernel(table, ux, uy)


def kernel(uv, layer1, layer2, layer3, layer4):
    tabs = [l[0].reshape(_N, -1).T for l in (layer1, layer2, layer3, layer4)]
    table = jnp.concatenate(tabs, axis=0)
    ux = uv[..., 0].reshape(-1)
    uy = uv[..., 1].reshape(-1)
    out = _sc_sample(table, ux, uy)
    return out.reshape(_B, _HG, _WG, _N).transpose(0, 3, 1, 2)


# speedup vs baseline: 2.8482x; 2.7270x over previous
import functools

import jax
import jax.numpy as jnp
from jax import lax
from jax.experimental import pallas as pl
from jax.experimental.pallas import tpu as pltpu
from jax.experimental.pallas import tpu_sc as plsc

_N = 96
_B, _HG, _WG = 4, 256, 256
_P = _B * _HG * _WG
_NW = 32
_PTS_PER_W = _P // _NW
_C = 32
_CHUNKS = _PTS_PER_W // _C
_TAPS = 16
_ROWS = _TAPS * _C
_BLK = 1024


def _sc_sample(table, ux, uy):
    mesh = plsc.VectorSubcoreMesh(core_axis_name="c", subcore_axis_name="s")

    @functools.partial(
        pl.kernel,
        out_type=jax.ShapeDtypeStruct((_P * _N,), jnp.float32),
        mesh=mesh,
        compiler_params=pltpu.CompilerParams(use_tc_tiling_on_sc=False),
        scratch_types=[
            pltpu.VMEM((_BLK,), jnp.float32),
            pltpu.VMEM((_BLK,), jnp.float32),
            pltpu.VMEM((_ROWS,), jnp.int32),
            pltpu.VMEM((_ROWS,), jnp.int32),
            pltpu.VMEM((_ROWS,), jnp.float32),
            pltpu.VMEM((_ROWS,), jnp.float32),
            pltpu.VMEM((_ROWS, _N), jnp.float32),
            pltpu.VMEM((_ROWS, _N), jnp.float32),
            pltpu.VMEM((_C * _N,), jnp.float32),
            pltpu.VMEM((_C * _N,), jnp.float32),
            pltpu.SemaphoreType.DMA,
            pltpu.SemaphoreType.DMA,
            pltpu.SemaphoreType.DMA,
            pltpu.SemaphoreType.DMA,
        ],
    )
    def tex_kernel(table_hbm, ux_hbm, uy_hbm, out_hbm,
                   ux_v, uy_v, idx_a, idx_b, w_a, w_b, taps_a, taps_b,
                   out_a, out_b, sem_a, sem_b, osem_a, osem_b):
        wid = lax.axis_index("s") * 2 + lax.axis_index("c")
        pbase = wid * _PTS_PER_W

        def stage(i, blkpos, idx_v, w_v):
            iota = lax.iota(jnp.int32, 16)
            lane_l = jnp.right_shift(iota, 2)
            wpitch_v = jnp.right_shift(iota * 0 + 512, lane_l)
            wm2_v = wpitch_v - 2
            sx_v = (wpitch_v - 1).astype(jnp.float32) * 0.5
            off_v = 349525 - jnp.right_shift(iota * 0 + 349525, 2 * lane_l)
            dx_v = jnp.bitwise_and(iota, 1)
            dy_v = jnp.bitwise_and(jnp.right_shift(iota, 1), 1)
            maskx = dx_v == 1
            masky = dy_v == 1

            for g in range(_C // 16):
                xs = ux_v[pl.ds(blkpos + g * 16, 16)]
                ys = uy_v[pl.ds(blkpos + g * 16, 16)]
                for pp in range(16):
                    p = g * 16 + pp
                    fx = (xs[pp] + 1.0) * sx_v
                    fy = (ys[pp] + 1.0) * sx_v
                    x0 = jnp.minimum(fx.astype(jnp.int32), wm2_v)
                    y0 = jnp.minimum(fy.astype(jnp.int32), wm2_v)
                    wx1 = fx - x0.astype(jnp.float32)
                    wy1 = fy - y0.astype(jnp.float32)
                    wxs = jnp.where(maskx, wx1, 1.0 - wx1)
                    wys = jnp.where(masky, wy1, 1.0 - wy1)
                    idx_v[pl.ds(p * _TAPS, 16)] = (
                        (y0 + dy_v) * wpitch_v + (x0 + dx_v) + off_v)
                    w_v[pl.ds(p * _TAPS, 16)] = wxs * wys

        def copies(idx_v, taps_v, sem):
            return [
                pltpu.make_async_copy(table_hbm.at[idx_v.at[pl.ds(j * 128, 128)]],
                                      taps_v.at[pl.ds(j * 128, 128)],
                                      sem)
                for j in range(_ROWS // 128)
            ]

        def issue(idx_v, taps_v, sem):
            for cp in copies(idx_v, taps_v, sem):
                cp.start()

        def drain(idx_v, taps_v, sem):
            for cp in copies(idx_v, taps_v, sem):
                cp.wait()

        def out_copy(i, out_v, osem):
            return pltpu.make_async_copy(
                out_v, out_hbm.at[pl.ds((pbase + i * _C) * _N, _C * _N)], osem)

        def accumulate(i, w_v, taps_v, out_v, osem):
            def point_body(q, cc):
                for pu in range(4):
                    p = q * 4 + pu
                    wv = w_v[pl.ds(p * _TAPS, 16)]
                    accs = [None] * (_N // 16)
                    for t in range(_TAPS):
                        w = wv[t]
                        for k in range(_N // 16):
                            term = w * taps_v[p * _TAPS + t, pl.ds(k * 16, 16)]
                            accs[k] = term if t == 0 else accs[k] + term
                    for k in range(_N // 16):
                        out_v[pl.ds(p * _N + k * 16, 16)] = accs[k]
                return cc

            lax.fori_loop(0, _C // 4, point_body, 0)
            out_copy(i, out_v, osem).start()

        pltpu.sync_copy(ux_hbm.at[pl.ds(pbase, _BLK)], ux_v)
        pltpu.sync_copy(uy_hbm.at[pl.ds(pbase, _BLK)], uy_v)
        stage(0, 0, idx_a, w_a)
        issue(idx_a, taps_a, sem_a)

        def body(j, carry):
            ia = 2 * j
            ib = 2 * j + 1
            jm = jnp.bitwise_and(j, 15)
            bpos_a = jm * (2 * _C)
            bpos_b = bpos_a + _C

            stage(ib, bpos_b, idx_b, w_b)
            issue(idx_b, taps_b, sem_b)
            drain(idx_a, taps_a, sem_a)

            @pl.when(j > 0)
            def _():
                out_copy(ia, out_a, osem_a).wait()

            accumulate(ia, w_a, taps_a, out_a, osem_a)

            @pl.when(jnp.bitwise_and(j + 1, 15) == 0)
            def _():
                pltpu.sync_copy(
                    ux_hbm.at[pl.ds(pbase + (j + 1) * (2 * _C), _BLK)], ux_v)
                pltpu.sync_copy(
                    uy_hbm.at[pl.ds(pbase + (j + 1) * (2 * _C), _BLK)], uy_v)

            @pl.when(ib + 1 < _CHUNKS)
            def _():
                stage(ib + 1, jnp.bitwise_and(j + 1, 15) * (2 * _C), idx_a, w_a)
                issue(idx_a, taps_a, sem_a)

            drain(idx_b, taps_b, sem_b)

            @pl.when(j > 0)
            def _():
                out_copy(ib, out_b, osem_b).wait()

            accumulate(ib, w_b, taps_b, out_b, osem_b)
            return carry

        lax.fori_loop(0, _CHUNKS // 2, body, 0)
        out_copy(_CHUNKS - 2, out_a, osem_a).wait()
        out_copy(_CHUNKS - 1, out_b, osem_b).wait()

    return tex_kernel(table, ux, uy)


def kernel(uv, layer1, layer2, layer3, layer4):
    tabs = [l[0].reshape(_N, -1).T for l in (layer1, layer2, layer3, layer4)]
    table = jnp.concatenate(tabs, axis=0)
    ux = uv[..., 0].reshape(-1)
    uy = uv[..., 1].reshape(-1)
    out = _sc_sample(table, ux, uy)
    return out.reshape(_B, _HG, _WG, _N).transpose(0, 3, 1, 2)
